# baseline (speedup 1.0000x reference)
import jax
import jax.numpy as jnp
from jax import lax
from jax.experimental import pallas as pl
from jax.experimental.pallas import tpu as pltpu
from jax.experimental.pallas import tpu_sc as plsc

F32 = jnp.float32
I32 = jnp.int32
U16 = jnp.uint16
BF16 = jnp.bfloat16

N = 10000
NP = 10240
E = 320000
H = 128
HW = 64
G = 256
CT = 16
TILES = 32
NCH = 80
ECH = E // 128
TCH = ECH - 31 * NCH
CHUNK = 128
NBUF = 8
RS = NP // 16
BT = 2048


def _mesh():
    return plsc.VectorSubcoreMesh(core_axis_name="c", subcore_axis_name="s")


def _load_idx(src3, plane, padt, pad_plane, buf, wid):
    @pl.when(wid < TILES - 1)
    def _full():
        pltpu.sync_copy(src3.at[plane, pl.ds(wid * NCH, NCH)], buf)

    @pl.when(wid == TILES - 1)
    def _tail():
        pltpu.sync_copy(src3.at[plane, pl.ds((TILES - 1) * NCH, TCH)],
                        buf.at[pl.ds(0, TCH)])
        pltpu.sync_copy(padt.at[pad_plane], buf.at[pl.ds(TCH, NCH - TCH)])


def _sc_embed_count_body(nt, an, ei3, bond3, padt, h0, cnt,
                         an_v, rows_v, bondb, dstb, o0, o1, zb, cnt_sh,
                         sem, cs0, cs1):
    cc = lax.axis_index("c")
    ss = lax.axis_index("s")
    wid = cc * 16 + ss
    zero16 = jnp.zeros((16,), F32)
    ones16 = jnp.ones((16,), F32)
    iota16 = lax.iota(I32, 16)

    for i in range(128):
        zb[i] = zero16
        o0[i] = zero16
        o1[i] = zero16
    for k in range(5):
        pltpu.sync_copy(zb, cnt_sh.at[pl.ds(ss * RS + k * 128, 128)])

    pltpu.sync_copy(an.at[wid], an_v)
    for k in range(5):
        pltpu.async_copy(nt.at[an_v.at[k]], rows_v, sem).wait()
        pltpu.sync_copy(rows_v, h0.at[pl.ds(wid * 320 + k * 64, 64)])

    _load_idx(bond3, 0, padt, 2, bondb, wid)
    _load_idx(ei3, 1, padt, 1, dstb, wid)
    plsc.subcore_barrier()

    bufs = (o0, o1)
    sems = (cs0, cs1)
    cd = [None, None]
    prev_pairs = [None, None]
    for ch in range(NCH):
        b = ch % 2
        if cd[b] is not None:
            cd[b].wait()
            for i0, b16 in prev_pairs[b]:
                plsc.store_scatter(bufs[b], [i0, b16], zero16)
        pairs = []
        for v in range(8):
            b16 = bondb[ch, pl.ds(v * 16, 16)]
            i0 = iota16 + v * 16
            pairs.append((i0, b16))
            plsc.store_scatter(bufs[b], [i0, b16], ones16)
        prev_pairs[b] = pairs
        cd[b] = pltpu.async_copy(bufs[b], cnt_sh.at[dstb.at[ch]], sems[b],
                                 add=True)
    cd[0].wait()
    cd[1].wait()
    plsc.subcore_barrier()
    pltpu.sync_copy(cnt_sh.at[pl.ds(ss * RS, RS)], cnt.at[cc, pl.ds(ss * RS, RS)])


def _sc_embed_count(nt, anp, ei3, bond3, padt):
    return pl.kernel(
        _sc_embed_count_body,
        out_type=(
            jax.ShapeDtypeStruct((NP, H), BF16),
            jax.ShapeDtypeStruct((2, NP, CT), F32),
        ),
        mesh=_mesh(),
        compiler_params=pltpu.CompilerParams(
            use_tc_tiling_on_sc=False, needs_layout_passes=False),
        scratch_types=[
            pltpu.VMEM((5, 64), I32),
            pltpu.VMEM((64, H), BF16),
            pltpu.VMEM((NCH, CHUNK), I32),
            pltpu.VMEM((NCH, CHUNK), I32),
            pltpu.VMEM((CHUNK, CT), F32),
            pltpu.VMEM((CHUNK, CT), F32),
            pltpu.VMEM((128, CT), F32),
            pltpu.VMEM_SHARED((NP, CT), F32),
            pltpu.SemaphoreType.DMA,
            pltpu.SemaphoreType.DMA,
            pltpu.SemaphoreType.DMA,
        ],
    )(nt, anp, ei3, bond3, padt)


def _sc_spmv_body(tab, ei3, padt, out,
                  srcb, dstb, rows, zb, agg_sh, gsems, ssems):
    cc = lax.axis_index("c")
    ss = lax.axis_index("s")
    wid = cc * 16 + ss

    _load_idx(ei3, 0, padt, 0, srcb, wid)
    _load_idx(ei3, 1, padt, 1, dstb, wid)
    zero32 = jnp.zeros((32,), BF16)
    for i in range(64):
        for j in range(4):
            zb[i, pl.ds(j * 32, 32)] = zero32
    for k in range(10):
        pltpu.sync_copy(zb, agg_sh.at[pl.ds(ss * RS + k * 64, 64)])
    plsc.subcore_barrier()

    gd = [None] * NBUF
    sd = [None] * NBUF

    def gather(c, b):
        return pltpu.async_copy(tab.at[srcb.at[c]], rows[b], gsems[b])

    for c in range(NBUF):
        gd[c] = gather(c, c)
    for c in range(NCH):
        b = c % NBUF
        gd[b].wait()
        sd[b] = pltpu.async_copy(rows[b], agg_sh.at[dstb.at[c]],
                                 ssems[b], add=True)
        n = c + NBUF
        if n < NCH:
            sd[b].wait()
            gd[b] = gather(n, b)
    for b in range(NBUF):
        sd[b].wait()
    plsc.subcore_barrier()
    pltpu.sync_copy(agg_sh.at[pl.ds(ss * RS, RS)], out.at[cc, pl.ds(ss * RS, RS)])


def _sc_spmv(tab, ei3, padt):
    return pl.kernel(
        _sc_spmv_body,
        out_type=jax.ShapeDtypeStruct((2, NP, H), BF16),
        mesh=_mesh(),
        compiler_params=pltpu.CompilerParams(use_tc_tiling_on_sc=False),
        scratch_types=[
            pltpu.VMEM((NCH, CHUNK), I32),
            pltpu.VMEM((NCH, CHUNK), I32),
            [pltpu.VMEM((CHUNK, H), BF16)] * NBUF,
            pltpu.VMEM((64, H), BF16),
            pltpu.VMEM_SHARED((NP, H), BF16),
            [pltpu.SemaphoreType.DMA] * NBUF,
            [pltpu.SemaphoreType.DMA] * NBUF,
        ],
    )(tab, ei3, padt)


def _tc_layer_body(a, c, btp, w, b, out):
    z = a[0].astype(F32) + a[1].astype(F32)
    z = z + jnp.dot(c[0] + c[1], btp[...], preferred_element_type=F32)
    h = jnp.maximum(jnp.dot(z, w[...], preferred_element_type=F32) + b[...], 0.0)
    out[...] = h.astype(BF16)


def _tc_layer(agg, cnt, btp, w, b):
    return pl.pallas_call(
        _tc_layer_body,
        grid=(NP // BT,),
        in_specs=[
            pl.BlockSpec((2, BT, H), lambda i: (0, i, 0)),
            pl.BlockSpec((2, BT, CT), lambda i: (0, i, 0)),
            pl.BlockSpec((CT, H), lambda i: (0, 0)),
            pl.BlockSpec((H, H), lambda i: (0, 0)),
            pl.BlockSpec((1, H), lambda i: (0, 0)),
        ],
        out_specs=pl.BlockSpec((BT, H), lambda i: (i, 0)),
        out_shape=jax.ShapeDtypeStruct((NP, H), BF16),
    )(agg, cnt, btp, w, b)


def _tc_final_body(a, c, btp, w, b, gid, out, acc, cn):
    i = pl.program_id(0)

    @pl.when(i == 0)
    def _init():
        acc[...] = jnp.zeros_like(acc)
        cn[...] = jnp.zeros_like(cn)

    z = a[0].astype(F32) + a[1].astype(F32)
    z = z + jnp.dot(c[0] + c[1], btp[...], preferred_element_type=F32)
    h3 = jnp.maximum(jnp.dot(z, w[...], preferred_element_type=F32) + b[...], 0.0)
    gv = gid[0, 0]
    mask = (lax.broadcasted_iota(I32, (G, BT), 0) == gv[None, :]).astype(F32)
    acc[...] += jnp.dot(mask, h3, preferred_element_type=F32)
    cn[...] += jnp.broadcast_to(jnp.sum(mask, axis=1, keepdims=True), (G, H))

    @pl.when(i == NP // BT - 1)
    def _fin():
        out[...] = acc[...] / jnp.maximum(cn[...], 1.0)


def _tc_final(agg, cnt, btp, w, b, gidp):
    return pl.pallas_call(
        _tc_final_body,
        grid=(NP // BT,),
        in_specs=[
            pl.BlockSpec((2, BT, H), lambda i: (0, i, 0)),
            pl.BlockSpec((2, BT, CT), lambda i: (0, i, 0)),
            pl.BlockSpec((CT, H), lambda i: (0, 0)),
            pl.BlockSpec((H, H), lambda i: (0, 0)),
            pl.BlockSpec((1, H), lambda i: (0, 0)),
            pl.BlockSpec((1, 1, BT), lambda i: (i, 0, 0)),
        ],
        out_specs=pl.BlockSpec((G, H), lambda i: (0, 0)),
        out_shape=jax.ShapeDtypeStruct((G, H), F32),
        scratch_shapes=[pltpu.VMEM((G, H), F32), pltpu.VMEM((G, H), F32)],
    )(agg, cnt, btp, w, b, gidp)


def kernel(atomic_number, edge_index, bond_type, graph_ids,
           node_table, bond_table, Ws, bs):
    ei3 = edge_index.astype(I32).reshape(2, ECH, CHUNK)
    bond3 = bond_type.astype(I32).reshape(1, ECH, CHUNK)
    npad = (NCH - TCH) * CHUNK
    side = jnp.concatenate([
        atomic_number.astype(I32), jnp.zeros((NP - N,), I32),
        graph_ids.astype(I32), jnp.full((NP - N,), G, I32),
        jnp.tile(N + (jnp.arange(npad, dtype=I32) % (NP - N)), 2),
        jnp.zeros((npad,), I32),
    ])
    anp = side[:NP].reshape(TILES, 5, 64)
    gidp = side[NP:2 * NP].reshape(NP // BT, 1, BT)
    padt = side[2 * NP:].reshape(3, NCH - TCH, CHUNK)
    btp = jnp.pad(bond_table.astype(F32), ((0, CT - bond_table.shape[0]), (0, 0)))
    ntp = node_table.astype(BF16)

    h, cnt = _sc_embed_count(ntp, anp, ei3, bond3, padt)
    L = Ws.shape[0]
    for l in range(L - 1):
        agg = _sc_spmv(h, ei3, padt)
        h = _tc_layer(agg, cnt, btp, Ws[l], bs[l][None, :])
    agg = _sc_spmv(h, ei3, padt)
    return _tc_final(agg, cnt, btp, Ws[L - 1], bs[L - 1][None, :], gidp)


# speedup vs baseline: 1.0016x; 1.0016x over previous
import jax
import jax.numpy as jnp
from jax import lax
from jax.experimental import pallas as pl
from jax.experimental.pallas import tpu as pltpu
from jax.experimental.pallas import tpu_sc as plsc

F32 = jnp.float32
I32 = jnp.int32
BF16 = jnp.bfloat16

N = 10000
NP = 10240
E = 320000
H = 128
G = 256
CT = 16
TILES = 32
NCH = 80
ECH = E // 128
TCH = ECH - 31 * NCH
CHUNK = 128
NBUF = 8
RS = NP // 16
BT = 2048


def _mesh():
    return plsc.VectorSubcoreMesh(core_axis_name="c", subcore_axis_name="s")


def _load_idx(src3, plane, padt, pad_plane, buf, wid):
    @pl.when(wid < TILES - 1)
    def _full():
        pltpu.sync_copy(src3.at[plane, pl.ds(wid * NCH, NCH)], buf)

    @pl.when(wid == TILES - 1)
    def _tail():
        pltpu.sync_copy(src3.at[plane, pl.ds((TILES - 1) * NCH, TCH)],
                        buf.at[pl.ds(0, TCH)])
        pltpu.sync_copy(padt.at[pad_plane], buf.at[pl.ds(TCH, NCH - TCH)])


def _sc_embed_count_body(nt, an, ei3, bond3, padt, h0, cnt,
                         an_v, rows_v, bondb, dstb, o0, o1, zb, cnt_sh,
                         sem, cs0, cs1):
    cc = lax.axis_index("c")
    ss = lax.axis_index("s")
    wid = cc * 16 + ss
    zero16 = jnp.zeros((16,), F32)
    ones16 = jnp.ones((16,), F32)
    iota16 = lax.iota(I32, 16)

    for i in range(128):
        zb[i] = zero16
        o0[i] = zero16
        o1[i] = zero16
    for k in range(5):
        pltpu.sync_copy(zb, cnt_sh.at[pl.ds(ss * RS + k * 128, 128)])

    pltpu.sync_copy(an.at[wid], an_v)
    for k in range(5):
        pltpu.async_copy(nt.at[an_v.at[k]], rows_v, sem).wait()
        pltpu.sync_copy(rows_v, h0.at[pl.ds(wid * 320 + k * 64, 64)])

    _load_idx(bond3, 0, padt, 2, bondb, wid)
    _load_idx(ei3, 1, padt, 1, dstb, wid)
    plsc.subcore_barrier()

    bufs = (o0, o1)
    sems = (cs0, cs1)
    cd = [None, None]
    prev_pairs = [None, None]
    for ch in range(NCH):
        b = ch % 2
        if cd[b] is not None:
            cd[b].wait()
            for i0, b16 in prev_pairs[b]:
                plsc.store_scatter(bufs[b], [i0, b16], zero16)
        pairs = []
        for v in range(8):
            b16 = bondb[ch, pl.ds(v * 16, 16)]
            i0 = iota16 + v * 16
            pairs.append((i0, b16))
            plsc.store_scatter(bufs[b], [i0, b16], ones16)
        prev_pairs[b] = pairs
        cd[b] = pltpu.async_copy(bufs[b], cnt_sh.at[dstb.at[ch]], sems[b],
                                 add=True)
    cd[0].wait()
    cd[1].wait()
    plsc.subcore_barrier()
    pltpu.sync_copy(cnt_sh.at[pl.ds(ss * RS, RS)], cnt.at[cc, pl.ds(ss * RS, RS)])


def _sc_embed_count(nt, anp, ei3, bond3, padt):
    return pl.kernel(
        _sc_embed_count_body,
        out_type=(
            jax.ShapeDtypeStruct((NP, H), BF16),
            jax.ShapeDtypeStruct((2, NP, CT), F32),
        ),
        mesh=_mesh(),
        compiler_params=pltpu.CompilerParams(
            use_tc_tiling_on_sc=False, needs_layout_passes=False),
        scratch_types=[
            pltpu.VMEM((5, 64), I32),
            pltpu.VMEM((64, H), BF16),
            pltpu.VMEM((NCH, CHUNK), I32),
            pltpu.VMEM((NCH, CHUNK), I32),
            pltpu.VMEM((CHUNK, CT), F32),
            pltpu.VMEM((CHUNK, CT), F32),
            pltpu.VMEM((128, CT), F32),
            pltpu.VMEM_SHARED((NP, CT), F32),
            pltpu.SemaphoreType.DMA,
            pltpu.SemaphoreType.DMA,
            pltpu.SemaphoreType.DMA,
        ],
    )(nt, anp, ei3, bond3, padt)


def _sc_spmv_body(tab, ei3, padt, out,
                  srcb, dstb, rows, zb, agg_sh, gsems, ssems):
    cc = lax.axis_index("c")
    ss = lax.axis_index("s")
    wid = cc * 16 + ss

    _load_idx(ei3, 0, padt, 0, srcb, wid)
    _load_idx(ei3, 1, padt, 1, dstb, wid)
    zero32 = jnp.zeros((32,), BF16)
    for i in range(64):
        for j in range(4):
            zb[i, pl.ds(j * 32, 32)] = zero32
    for k in range(10):
        pltpu.sync_copy(zb, agg_sh.at[pl.ds(ss * RS + k * 64, 64)])
    plsc.subcore_barrier()

    gd = [None] * NBUF
    sd = [None] * NBUF

    def gather(c, b):
        return pltpu.async_copy(tab.at[srcb.at[c]], rows[b], gsems[b])

    for c in range(NBUF):
        gd[c] = gather(c, c)
    for c in range(NCH):
        b = c % NBUF
        gd[b].wait()
        sd[b] = pltpu.async_copy(rows[b], agg_sh.at[dstb.at[c]],
                                 ssems[b], add=True)
        n = c + NBUF
        if n < NCH:
            sd[b].wait()
            gd[b] = gather(n, b)
    for b in range(NBUF):
        sd[b].wait()
    plsc.subcore_barrier()
    pltpu.sync_copy(agg_sh.at[pl.ds(ss * RS, RS)], out.at[cc, pl.ds(ss * RS, RS)])


def _sc_spmv(tab, ei3, padt):
    return pl.kernel(
        _sc_spmv_body,
        out_type=jax.ShapeDtypeStruct((2, NP, H), BF16),
        mesh=_mesh(),
        compiler_params=pltpu.CompilerParams(use_tc_tiling_on_sc=False),
        scratch_types=[
            pltpu.VMEM((NCH, CHUNK), I32),
            pltpu.VMEM((NCH, CHUNK), I32),
            [pltpu.VMEM((CHUNK, H), BF16)] * NBUF,
            pltpu.VMEM((64, H), BF16),
            pltpu.VMEM_SHARED((NP, H), BF16),
            [pltpu.SemaphoreType.DMA] * NBUF,
            [pltpu.SemaphoreType.DMA] * NBUF,
        ],
    )(tab, ei3, padt)


def _tc_layer_body(a, c, btp, w, b, out):
    z = a[0].astype(F32) + a[1].astype(F32)
    z = z + jnp.dot(c[0] + c[1], btp[...], preferred_element_type=F32)
    h = jnp.maximum(jnp.dot(z, w[...], preferred_element_type=F32) + b[...], 0.0)
    out[...] = h.astype(BF16)


def _tc_layer(agg, cnt, btp, w, b):
    return pl.pallas_call(
        _tc_layer_body,
        grid=(NP // BT,),
        in_specs=[
            pl.BlockSpec((2, BT, H), lambda i: (0, i, 0)),
            pl.BlockSpec((2, BT, CT), lambda i: (0, i, 0)),
            pl.BlockSpec((CT, H), lambda i: (0, 0)),
            pl.BlockSpec((H, H), lambda i: (0, 0)),
            pl.BlockSpec((1, H), lambda i: (0, 0)),
        ],
        out_specs=pl.BlockSpec((BT, H), lambda i: (i, 0)),
        out_shape=jax.ShapeDtypeStruct((NP, H), BF16),
    )(agg, cnt, btp, w, b)


def _tc_final_body(a, c, btp, w, b, gid, out, acc, cn):
    i = pl.program_id(0)

    @pl.when(i == 0)
    def _init():
        acc[...] = jnp.zeros_like(acc)
        cn[...] = jnp.zeros_like(cn)

    z = a[0].astype(F32) + a[1].astype(F32)
    z = z + jnp.dot(c[0] + c[1], btp[...], preferred_element_type=F32)
    h3 = jnp.maximum(jnp.dot(z, w[...], preferred_element_type=F32) + b[...], 0.0)
    gv = gid[0, 0]
    mask = (lax.broadcasted_iota(I32, (G, BT), 0) == gv[None, :]).astype(F32)
    acc[...] += jnp.dot(mask, h3, preferred_element_type=F32)
    cn[...] += jnp.broadcast_to(jnp.sum(mask, axis=1, keepdims=True), (G, H))

    @pl.when(i == NP // BT - 1)
    def _fin():
        out[...] = acc[...] / jnp.maximum(cn[...], 1.0)


def _tc_final(agg, cnt, btp, w, b, gidp):
    return pl.pallas_call(
        _tc_final_body,
        grid=(NP // BT,),
        in_specs=[
            pl.BlockSpec((2, BT, H), lambda i: (0, i, 0)),
            pl.BlockSpec((2, BT, CT), lambda i: (0, i, 0)),
            pl.BlockSpec((CT, H), lambda i: (0, 0)),
            pl.BlockSpec((H, H), lambda i: (0, 0)),
            pl.BlockSpec((1, H), lambda i: (0, 0)),
            pl.BlockSpec((1, 1, BT), lambda i: (i, 0, 0)),
        ],
        out_specs=pl.BlockSpec((G, H), lambda i: (0, 0)),
        out_shape=jax.ShapeDtypeStruct((G, H), F32),
        scratch_shapes=[pltpu.VMEM((G, H), F32), pltpu.VMEM((G, H), F32)],
    )(agg, cnt, btp, w, b, gidp)


def kernel(atomic_number, edge_index, bond_type, graph_ids,
           node_table, bond_table, Ws, bs):
    ei3 = edge_index.astype(I32).reshape(2, ECH, CHUNK)
    bond3 = bond_type.astype(I32).reshape(1, ECH, CHUNK)
    npad = (NCH - TCH) * CHUNK
    side = jnp.concatenate([
        atomic_number.astype(I32), jnp.zeros((NP - N,), I32),
        graph_ids.astype(I32), jnp.full((NP - N,), G, I32),
        jnp.tile(N + (jnp.arange(npad, dtype=I32) % (NP - N)), 2),
        jnp.zeros((npad,), I32),
    ])
    anp = side[:NP].reshape(TILES, 5, 64)
    gidp = side[NP:2 * NP].reshape(NP // BT, 1, BT)
    padt = side[2 * NP:].reshape(3, NCH - TCH, CHUNK)
    btp = jnp.pad(bond_table.astype(F32), ((0, CT - bond_table.shape[0]), (0, 0)))
    ntp = node_table.astype(BF16)

    h, cnt = _sc_embed_count(ntp, anp, ei3, bond3, padt)
    L = Ws.shape[0]
    for l in range(L - 1):
        agg = _sc_spmv(h, ei3, padt)
        h = _tc_layer(agg, cnt, btp, Ws[l], bs[l][None, :])
    agg = _sc_spmv(h, ei3, padt)
    return _tc_final(agg, cnt, btp, Ws[L - 1], bs[L - 1][None, :], gidp)


# speedup vs baseline: 1.0050x; 1.0034x over previous
import jax
import jax.numpy as jnp
from jax import lax
from jax.experimental import pallas as pl
from jax.experimental.pallas import tpu as pltpu
from jax.experimental.pallas import tpu_sc as plsc

F32 = jnp.float32
I32 = jnp.int32
BF16 = jnp.bfloat16

N = 10000
NP = 10240
E = 320000
H = 128
G = 256
CT = 16
TILES = 32
NCH = 80
ECH = E // 128
TCH = ECH - 31 * NCH
CHUNK = 128
NBUF = 8
RS = NP // 16
BT = 2048


def _mesh():
    return plsc.VectorSubcoreMesh(core_axis_name="c", subcore_axis_name="s")


def _load_idx(src3, plane, padt, pad_plane, buf, wid):
    @pl.when(wid < TILES - 1)
    def _full():
        pltpu.sync_copy(src3.at[plane, pl.ds(wid * NCH, NCH)], buf)

    @pl.when(wid == TILES - 1)
    def _tail():
        pltpu.sync_copy(src3.at[plane, pl.ds((TILES - 1) * NCH, TCH)],
                        buf.at[pl.ds(0, TCH)])
        pltpu.sync_copy(padt.at[pad_plane], buf.at[pl.ds(TCH, NCH - TCH)])


def _sc_embed_count_body(nt, an, ei3, bond3, padt, h0, cnt,
                         an_v, rv0, rv1, bondb, dstb, obufs, zb, cnt_sh,
                         g0, g1, csems):
    cc = lax.axis_index("c")
    ss = lax.axis_index("s")
    wid = cc * 16 + ss
    zero16 = jnp.zeros((16,), F32)
    ones16 = jnp.ones((16,), F32)
    iota16 = lax.iota(I32, 16)

    for i in range(128):
        zb[i] = zero16
        for ob in obufs:
            ob[i] = zero16
    for k in range(5):
        pltpu.sync_copy(zb, cnt_sh.at[pl.ds(ss * RS + k * 128, 128)])

    pltpu.sync_copy(an.at[wid], an_v)
    rvs = (rv0, rv1)
    gsems = (g0, g1)
    gd = [None, None]
    gd[0] = pltpu.async_copy(nt.at[an_v.at[0]], rv0, g0)
    for k in range(5):
        b = k % 2
        gd[b].wait()
        if k + 1 < 5:
            nb = (k + 1) % 2
            gd[nb] = pltpu.async_copy(nt.at[an_v.at[k + 1]], rvs[nb], gsems[nb])
        pltpu.sync_copy(rvs[b], h0.at[pl.ds(wid * 320 + k * 64, 64)])

    _load_idx(bond3, 0, padt, 2, bondb, wid)
    _load_idx(ei3, 1, padt, 1, dstb, wid)
    plsc.subcore_barrier()

    nob = len(obufs)
    cd = [None] * nob
    prev_pairs = [None] * nob
    for ch in range(NCH):
        b = ch % nob
        if cd[b] is not None:
            cd[b].wait()
            for i0, b16 in prev_pairs[b]:
                plsc.store_scatter(obufs[b], [i0, b16], zero16)
        pairs = []
        for v in range(8):
            b16 = bondb[ch, pl.ds(v * 16, 16)]
            i0 = iota16 + v * 16
            pairs.append((i0, b16))
            plsc.store_scatter(obufs[b], [i0, b16], ones16)
        prev_pairs[b] = pairs
        cd[b] = pltpu.async_copy(obufs[b], cnt_sh.at[dstb.at[ch]], csems[b],
                                 add=True)
    for d in cd:
        d.wait()
    plsc.subcore_barrier()
    pltpu.sync_copy(cnt_sh.at[pl.ds(ss * RS, RS)], cnt.at[cc, pl.ds(ss * RS, RS)])


def _sc_embed_count(nt, anp, ei3, bond3, padt):
    return pl.kernel(
        _sc_embed_count_body,
        out_type=(
            jax.ShapeDtypeStruct((NP, H), BF16),
            jax.ShapeDtypeStruct((2, NP, CT), F32),
        ),
        mesh=_mesh(),
        compiler_params=pltpu.CompilerParams(
            use_tc_tiling_on_sc=False, needs_layout_passes=False),
        scratch_types=[
            pltpu.VMEM((5, 64), I32),
            pltpu.VMEM((64, H), BF16),
            pltpu.VMEM((64, H), BF16),
            pltpu.VMEM((NCH, CHUNK), I32),
            pltpu.VMEM((NCH, CHUNK), I32),
            [pltpu.VMEM((CHUNK, CT), F32)] * 4,
            pltpu.VMEM((128, CT), F32),
            pltpu.VMEM_SHARED((NP, CT), F32),
            pltpu.SemaphoreType.DMA,
            pltpu.SemaphoreType.DMA,
            [pltpu.SemaphoreType.DMA] * 4,
        ],
    )(nt, anp, ei3, bond3, padt)


def _sc_spmv_body(tab, ei3, padt, out,
                  srcb, dstb, rows, zb, agg_sh, gsems, ssems):
    cc = lax.axis_index("c")
    ss = lax.axis_index("s")
    wid = cc * 16 + ss

    _load_idx(ei3, 0, padt, 0, srcb, wid)
    _load_idx(ei3, 1, padt, 1, dstb, wid)
    zero32 = jnp.zeros((32,), BF16)
    for i in range(64):
        for j in range(4):
            zb[i, pl.ds(j * 32, 32)] = zero32
    for k in range(10):
        pltpu.sync_copy(zb, agg_sh.at[pl.ds(ss * RS + k * 64, 64)])
    plsc.subcore_barrier()

    gd = [None] * NBUF
    sd = [None] * NBUF

    def gather(c, b):
        return pltpu.async_copy(tab.at[srcb.at[c]], rows[b], gsems[b])

    for c in range(NBUF):
        gd[c] = gather(c, c)
    for c in range(NCH):
        b = c % NBUF
        gd[b].wait()
        sd[b] = pltpu.async_copy(rows[b], agg_sh.at[dstb.at[c]],
                                 ssems[b], add=True)
        n = c + NBUF
        if n < NCH:
            sd[b].wait()
            gd[b] = gather(n, b)
    for b in range(NBUF):
        sd[b].wait()
    plsc.subcore_barrier()
    pltpu.sync_copy(agg_sh.at[pl.ds(ss * RS, RS)], out.at[cc, pl.ds(ss * RS, RS)])


def _sc_spmv(tab, ei3, padt):
    return pl.kernel(
        _sc_spmv_body,
        out_type=jax.ShapeDtypeStruct((2, NP, H), BF16),
        mesh=_mesh(),
        compiler_params=pltpu.CompilerParams(use_tc_tiling_on_sc=False),
        scratch_types=[
            pltpu.VMEM((NCH, CHUNK), I32),
            pltpu.VMEM((NCH, CHUNK), I32),
            [pltpu.VMEM((CHUNK, H), BF16)] * NBUF,
            pltpu.VMEM((64, H), BF16),
            pltpu.VMEM_SHARED((NP, H), BF16),
            [pltpu.SemaphoreType.DMA] * NBUF,
            [pltpu.SemaphoreType.DMA] * NBUF,
        ],
    )(tab, ei3, padt)


def _tc_layer_body(a, c, btp, w, b, out):
    z = a[0].astype(F32) + a[1].astype(F32)
    z = z + jnp.dot(c[0] + c[1], btp[...], preferred_element_type=F32)
    h = jnp.maximum(jnp.dot(z, w[...], preferred_element_type=F32) + b[...], 0.0)
    out[...] = h.astype(BF16)


def _tc_layer(agg, cnt, btp, w, b):
    return pl.pallas_call(
        _tc_layer_body,
        grid=(NP // BT,),
        in_specs=[
            pl.BlockSpec((2, BT, H), lambda i: (0, i, 0)),
            pl.BlockSpec((2, BT, CT), lambda i: (0, i, 0)),
            pl.BlockSpec((CT, H), lambda i: (0, 0)),
            pl.BlockSpec((H, H), lambda i: (0, 0)),
            pl.BlockSpec((1, H), lambda i: (0, 0)),
        ],
        out_specs=pl.BlockSpec((BT, H), lambda i: (i, 0)),
        out_shape=jax.ShapeDtypeStruct((NP, H), BF16),
    )(agg, cnt, btp, w, b)


def _tc_final_body(a, c, btp, w, b, gid, out, acc, cn):
    i = pl.program_id(0)

    @pl.when(i == 0)
    def _init():
        acc[...] = jnp.zeros_like(acc)
        cn[...] = jnp.zeros_like(cn)

    z = a[0].astype(F32) + a[1].astype(F32)
    z = z + jnp.dot(c[0] + c[1], btp[...], preferred_element_type=F32)
    h3 = jnp.maximum(jnp.dot(z, w[...], preferred_element_type=F32) + b[...], 0.0)
    gv = gid[0, 0]
    mask = (lax.broadcasted_iota(I32, (G, BT), 0) == gv[None, :]).astype(F32)
    acc[...] += jnp.dot(mask, h3, preferred_element_type=F32)
    cn[...] += jnp.broadcast_to(jnp.sum(mask, axis=1, keepdims=True), (G, H))

    @pl.when(i == NP // BT - 1)
    def _fin():
        out[...] = acc[...] / jnp.maximum(cn[...], 1.0)


def _tc_final(agg, cnt, btp, w, b, gidp):
    return pl.pallas_call(
        _tc_final_body,
        grid=(NP // BT,),
        in_specs=[
            pl.BlockSpec((2, BT, H), lambda i: (0, i, 0)),
            pl.BlockSpec((2, BT, CT), lambda i: (0, i, 0)),
            pl.BlockSpec((CT, H), lambda i: (0, 0)),
            pl.BlockSpec((H, H), lambda i: (0, 0)),
            pl.BlockSpec((1, H), lambda i: (0, 0)),
            pl.BlockSpec((1, 1, BT), lambda i: (i, 0, 0)),
        ],
        out_specs=pl.BlockSpec((G, H), lambda i: (0, 0)),
        out_shape=jax.ShapeDtypeStruct((G, H), F32),
        scratch_shapes=[pltpu.VMEM((G, H), F32), pltpu.VMEM((G, H), F32)],
    )(agg, cnt, btp, w, b, gidp)


def kernel(atomic_number, edge_index, bond_type, graph_ids,
           node_table, bond_table, Ws, bs):
    ei3 = edge_index.astype(I32).reshape(2, ECH, CHUNK)
    bond3 = bond_type.astype(I32).reshape(1, ECH, CHUNK)
    npad = (NCH - TCH) * CHUNK
    side = jnp.concatenate([
        atomic_number.astype(I32), jnp.zeros((NP - N,), I32),
        graph_ids.astype(I32), jnp.full((NP - N,), G, I32),
        jnp.tile(N + (jnp.arange(npad, dtype=I32) % (NP - N)), 2),
        jnp.zeros((npad,), I32),
    ])
    anp = side[:NP].reshape(TILES, 5, 64)
    gidp = side[NP:2 * NP].reshape(NP // BT, 1, BT)
    padt = side[2 * NP:].reshape(3, NCH - TCH, CHUNK)
    btp = jnp.pad(bond_table.astype(F32), ((0, CT - bond_table.shape[0]), (0, 0)))
    ntp = node_table.astype(BF16)

    h, cnt = _sc_embed_count(ntp, anp, ei3, bond3, padt)
    L = Ws.shape[0]
    for l in range(L - 1):
        agg = _sc_spmv(h, ei3, padt)
        h = _tc_layer(agg, cnt, btp, Ws[l], bs[l][None, :])
    agg = _sc_spmv(h, ei3, padt)
    return _tc_final(agg, cnt, btp, Ws[L - 1], bs[L - 1][None, :], gidp)
